# Initial kernel scaffold; baseline (speedup 1.0000x reference)
import jax
import jax.numpy as jnp
from jax.experimental import pallas as pl


def kernel(loc_pred, conf_pred, priors, gt_boxes, gt_labels):
    raise NotImplementedError("write your pallas kernel here")


# speedup vs baseline: 14.1660x; 14.1660x over previous
import functools

import jax
import jax.numpy as jnp
from jax import lax
from jax.experimental import pallas as pl

_SUB = 8
_LANE = 1152
_PPAD = _SUB * _LANE

_THRESHOLD = 0.5
_NEG_POS_RATIO = 3
_VAR0 = 0.1
_VAR1 = 0.2


def _mb_kernel(loc_ref, conf_ref, pri_ref, gtb_ref, gtl_ref, out_ref, *, P, G, C):
    b = pl.program_id(0)
    f32 = jnp.float32
    i32 = jnp.int32

    sub_i = lax.broadcasted_iota(i32, (_SUB, _LANE), 0)
    lane_i = lax.broadcasted_iota(i32, (_SUB, _LANE), 1)
    pid = sub_i * _LANE + lane_i
    valid = pid < P

    pri = pri_ref[...]
    pcx, pcy, pw, ph = pri[0], pri[1], pri[2], pri[3]
    px1 = pcx - pw / 2.0
    py1 = pcy - ph / 2.0
    px2 = pcx + pw / 2.0
    py2 = pcy + ph / 2.0
    area_p = (px2 - px1) * (py2 - py1)

    neg1 = jnp.full((_SUB, _LANE), -1.0, f32)
    max_iou = neg1
    bgpp = jnp.zeros((_SUB, _LANE), i32)
    forced = jnp.full((_SUB, _LANE), -1, i32)
    ious = []
    for g in range(G):
        gx1 = gtb_ref[0, g, 0]
        gy1 = gtb_ref[0, g, 1]
        gx2 = gtb_ref[0, g, 2]
        gy2 = gtb_ref[0, g, 3]
        ltx = jnp.maximum(px1, gx1)
        lty = jnp.maximum(py1, gy1)
        rbx = jnp.minimum(px2, gx2)
        rby = jnp.minimum(py2, gy2)
        wx = jnp.clip(rbx - ltx, 0.0, None)
        wy = jnp.clip(rby - lty, 0.0, None)
        inter = wx * wy
        area_g = (gx2 - gx1) * (gy2 - gy1)
        iou_g = inter / (area_p + area_g - inter + 1e-10)
        iou_g = jnp.where(valid, iou_g, -1.0)
        ious.append(iou_g)
        better = iou_g > max_iou
        bgpp = jnp.where(better, g, bgpp)
        max_iou = jnp.where(better, iou_g, max_iou)

    big = jnp.int32(2 ** 30)
    for g in range(G):
        mx = jnp.max(ious[g])
        bp = jnp.min(jnp.where(ious[g] == mx, pid, big))
        forced = jnp.where(pid == bp, g, forced)

    above = max_iou >= _THRESHOLD
    matched = jnp.where(above, jnp.where(forced >= 0, forced, bgpp), -1)
    pos = matched >= 0
    num_pos_i = jnp.sum(pos.astype(i32))
    idx = jnp.maximum(matched, 0)

    mx1 = jnp.zeros((_SUB, _LANE), f32)
    my1 = jnp.zeros((_SUB, _LANE), f32)
    mx2 = jnp.zeros((_SUB, _LANE), f32)
    my2 = jnp.zeros((_SUB, _LANE), f32)
    mlab = jnp.zeros((_SUB, _LANE), f32)
    for g in range(G):
        m = idx == g
        mx1 = jnp.where(m, gtb_ref[0, g, 0], mx1)
        my1 = jnp.where(m, gtb_ref[0, g, 1], my1)
        mx2 = jnp.where(m, gtb_ref[0, g, 2], mx2)
        my2 = jnp.where(m, gtb_ref[0, g, 3], my2)
        mlab = jnp.where(m, gtl_ref[0, 0, g].astype(f32), mlab)

    bcx = (mx1 + mx2) / 2.0
    bcy = (my1 + my2) / 2.0
    bw = mx2 - mx1
    bh = my2 - my1
    g_cx = (bcx - pcx) / (_VAR0 * pw)
    g_cy = (bcy - pcy) / (_VAR0 * ph)
    w_safe = jnp.where(pos, bw, 1.0)
    h_safe = jnp.where(pos, bh, 1.0)
    g_w = jnp.log(jnp.maximum(w_safe / pw, 1e-10)) / _VAR1
    g_h = jnp.log(jnp.maximum(h_safe / ph, 1e-10)) / _VAR1

    loc = loc_ref[0]
    loc_loss = jnp.float32(0.0)
    for t, lrow in ((g_cx, loc[0]), (g_cy, loc[1]), (g_w, loc[2]), (g_h, loc[3])):
        d = lrow - jnp.where(pos, t, 0.0)
        ad = jnp.abs(d)
        sl1 = jnp.where(ad < 1.0, 0.5 * d * d, ad - 0.5)
        loc_loss = loc_loss + jnp.sum(jnp.where(pos, sl1, 0.0))

    tgt = jnp.where(pos, mlab.astype(i32) - 1, 0)
    conf = conf_ref[0]
    cmax = conf[0]
    for c in range(1, C):
        cmax = jnp.maximum(cmax, conf[c])
    s = jnp.zeros((_SUB, _LANE), f32)
    x_tgt = jnp.zeros((_SUB, _LANE), f32)
    for c in range(C):
        s = s + jnp.exp(conf[c] - cmax)
        x_tgt = jnp.where(tgt == c, conf[c], x_tgt)
    ce = cmax + jnp.log(s) - x_tgt

    sum_pos_ce = jnp.sum(jnp.where(pos, ce, 0.0))
    neg_loss = jnp.where(pos | (~valid), 0.0, ce)

    n_strict = jnp.sum((neg_loss > 0.0).astype(i32))
    k = jnp.minimum(num_pos_i * _NEG_POS_RATIO, P - 1)
    kp = jnp.minimum(k, n_strict)
    vbits = lax.bitcast_convert_type(neg_loss, i32)

    def bs_body(_, carry):
        lo, hi = carry
        mid = lo + (hi - lo) // 2
        c = jnp.sum((vbits > mid).astype(i32))
        go = c >= kp
        return jnp.where(go, mid + 1, lo), jnp.where(go, hi, mid)

    lo, _ = lax.fori_loop(0, 31, bs_body, (jnp.int32(0), jnp.int32(2 ** 31 - 1)))
    t = lax.bitcast_convert_type(lo, f32)
    gt_mask = neg_loss > t
    c1 = jnp.sum(gt_mask.astype(i32))
    sum_gt = jnp.sum(jnp.where(gt_mask, neg_loss, 0.0))
    sum_topk = jnp.where(kp > 0, sum_gt + (kp - c1).astype(f32) * t, 0.0)

    conf_loss = sum_pos_ce + sum_topk
    num_conf = (num_pos_i + kp).astype(f32)

    o_sub = lax.broadcasted_iota(i32, (8, 128), 0)
    o_lane = lax.broadcasted_iota(i32, (8, 128), 1)
    r0 = o_sub == 0
    contrib = (
        jnp.where(r0 & (o_lane == 0), loc_loss, 0.0)
        + jnp.where(r0 & (o_lane == 1), (num_pos_i * 4).astype(f32), 0.0)
        + jnp.where(r0 & (o_lane == 2), conf_loss, 0.0)
        + jnp.where(r0 & (o_lane == 3), num_conf, 0.0)
    )

    @pl.when(b == 0)
    def _():
        out_ref[...] = jnp.zeros((8, 128), f32)

    out_ref[...] += contrib


def kernel(loc_pred, conf_pred, priors, gt_boxes, gt_labels):
    B, P, C = conf_pred.shape
    G = gt_boxes.shape[1]
    pad = _PPAD - P

    loc_t = jnp.pad(
        jnp.transpose(loc_pred, (0, 2, 1)), ((0, 0), (0, 0), (0, pad))
    ).reshape(B, 4, _SUB, _LANE)
    conf_t = jnp.pad(
        jnp.transpose(conf_pred, (0, 2, 1)), ((0, 0), (0, 0), (0, pad))
    ).reshape(B, C, _SUB, _LANE)
    pri_t = jnp.pad(
        priors.T, ((0, 0), (0, pad)), constant_values=1.0
    ).reshape(4, _SUB, _LANE)
    gtl = gt_labels.astype(jnp.int32).reshape(B, 1, G)

    out = pl.pallas_call(
        functools.partial(_mb_kernel, P=P, G=G, C=C),
        grid=(B,),
        in_specs=[
            pl.BlockSpec((1, 4, _SUB, _LANE), lambda b: (b, 0, 0, 0)),
            pl.BlockSpec((1, C, _SUB, _LANE), lambda b: (b, 0, 0, 0)),
            pl.BlockSpec((4, _SUB, _LANE), lambda b: (0, 0, 0)),
            pl.BlockSpec((1, G, 4), lambda b: (b, 0, 0)),
            pl.BlockSpec((1, 1, G), lambda b: (b, 0, 0)),
        ],
        out_specs=pl.BlockSpec((8, 128), lambda b: (0, 0)),
        out_shape=jax.ShapeDtypeStruct((8, 128), jnp.float32),
    )(loc_t, conf_t, pri_t, gt_boxes, gtl)
    return out[0, 0] / out[0, 1] + out[0, 2] / out[0, 3]


# speedup vs baseline: 21.7464x; 1.5351x over previous
import functools

import jax
import jax.numpy as jnp
from jax import lax
from jax.experimental import pallas as pl

_SUB = 8
_LANE = 1152
_PPAD = _SUB * _LANE

_THRESHOLD = 0.5
_NEG_POS_RATIO = 3
_VAR0 = 0.1
_VAR1 = 0.2


def _stage1_kernel(loc_ref, conf_ref, pri_ref, gtb_ref, gtl_ref,
                   neg_ref, scal_ref, *, P, G, C):
    f32 = jnp.float32
    i32 = jnp.int32

    sub_i = lax.broadcasted_iota(i32, (_SUB, _LANE), 0)
    lane_i = lax.broadcasted_iota(i32, (_SUB, _LANE), 1)
    pid = sub_i * _LANE + lane_i
    valid = pid < P

    pri = pri_ref[...]
    pcx, pcy, pw, ph = pri[0], pri[1], pri[2], pri[3]
    px1 = pcx - pw / 2.0
    py1 = pcy - ph / 2.0
    px2 = pcx + pw / 2.0
    py2 = pcy + ph / 2.0
    area_p = (px2 - px1) * (py2 - py1)

    max_iou = jnp.full((_SUB, _LANE), -1.0, f32)
    bgpp = jnp.zeros((_SUB, _LANE), i32)
    forced = jnp.full((_SUB, _LANE), -1, i32)
    ious = []
    for g in range(G):
        gx1 = gtb_ref[0, g, 0]
        gy1 = gtb_ref[0, g, 1]
        gx2 = gtb_ref[0, g, 2]
        gy2 = gtb_ref[0, g, 3]
        ltx = jnp.maximum(px1, gx1)
        lty = jnp.maximum(py1, gy1)
        rbx = jnp.minimum(px2, gx2)
        rby = jnp.minimum(py2, gy2)
        wx = jnp.clip(rbx - ltx, 0.0, None)
        wy = jnp.clip(rby - lty, 0.0, None)
        inter = wx * wy
        area_g = (gx2 - gx1) * (gy2 - gy1)
        iou_g = inter / (area_p + area_g - inter + 1e-10)
        iou_g = jnp.where(valid, iou_g, -1.0)
        ious.append(iou_g)
        better = iou_g > max_iou
        bgpp = jnp.where(better, g, bgpp)
        max_iou = jnp.where(better, iou_g, max_iou)

    big = jnp.int32(2 ** 30)
    for g in range(G):
        mx = jnp.max(ious[g])
        bp = jnp.min(jnp.where(ious[g] == mx, pid, big))
        forced = jnp.where(pid == bp, g, forced)

    above = max_iou >= _THRESHOLD
    matched = jnp.where(above, jnp.where(forced >= 0, forced, bgpp), -1)
    pos = matched >= 0
    num_pos_i = jnp.sum(pos.astype(i32))
    idx = jnp.maximum(matched, 0)

    mx1 = jnp.zeros((_SUB, _LANE), f32)
    my1 = jnp.zeros((_SUB, _LANE), f32)
    mx2 = jnp.zeros((_SUB, _LANE), f32)
    my2 = jnp.zeros((_SUB, _LANE), f32)
    mlab = jnp.zeros((_SUB, _LANE), f32)
    for g in range(G):
        m = idx == g
        mx1 = jnp.where(m, gtb_ref[0, g, 0], mx1)
        my1 = jnp.where(m, gtb_ref[0, g, 1], my1)
        mx2 = jnp.where(m, gtb_ref[0, g, 2], mx2)
        my2 = jnp.where(m, gtb_ref[0, g, 3], my2)
        mlab = jnp.where(m, gtl_ref[0, 0, g].astype(f32), mlab)

    bcx = (mx1 + mx2) / 2.0
    bcy = (my1 + my2) / 2.0
    bw = mx2 - mx1
    bh = my2 - my1
    g_cx = (bcx - pcx) / (_VAR0 * pw)
    g_cy = (bcy - pcy) / (_VAR0 * ph)
    w_safe = jnp.where(pos, bw, 1.0)
    h_safe = jnp.where(pos, bh, 1.0)
    g_w = jnp.log(jnp.maximum(w_safe / pw, 1e-10)) / _VAR1
    g_h = jnp.log(jnp.maximum(h_safe / ph, 1e-10)) / _VAR1

    loc = loc_ref[0]
    loc_loss = jnp.float32(0.0)
    for t, lrow in ((g_cx, loc[0]), (g_cy, loc[1]), (g_w, loc[2]), (g_h, loc[3])):
        d = lrow - jnp.where(pos, t, 0.0)
        ad = jnp.abs(d)
        sl1 = jnp.where(ad < 1.0, 0.5 * d * d, ad - 0.5)
        loc_loss = loc_loss + jnp.sum(jnp.where(pos, sl1, 0.0))

    tgt = jnp.where(pos, mlab.astype(i32) - 1, 0)
    conf = conf_ref[0]
    cmax = conf[0]
    for c in range(1, C):
        cmax = jnp.maximum(cmax, conf[c])
    s = jnp.zeros((_SUB, _LANE), f32)
    x_tgt = jnp.zeros((_SUB, _LANE), f32)
    for c in range(C):
        s = s + jnp.exp(conf[c] - cmax)
        x_tgt = jnp.where(tgt == c, conf[c], x_tgt)
    ce = cmax + jnp.log(s) - x_tgt

    sum_pos_ce = jnp.sum(jnp.where(pos, ce, 0.0))
    neg_loss = jnp.where(pos | (~valid), 0.0, ce)

    neg_ref[...] = neg_loss.reshape(1, _SUB, _LANE)
    o_lane = lax.broadcasted_iota(i32, (1, 1, 128), 2)
    scal_ref[...] = (
        jnp.where(o_lane == 0, loc_loss, 0.0)
        + jnp.where(o_lane == 1, num_pos_i.astype(f32), 0.0)
        + jnp.where(o_lane == 2, sum_pos_ce, 0.0)
    )


def _stage2_kernel(neg_ref, scal_ref, out_ref, *, B, P):
    f32 = jnp.float32
    i32 = jnp.int32

    v = neg_ref[...]
    vb = lax.bitcast_convert_type(v, i32)
    scal = scal_ref[...]
    loc_v = scal[:, 0:1]
    np_v = scal[:, 1:2]
    spce_v = scal[:, 2:3]

    np_i = np_v.astype(i32)
    n_strict = jnp.sum((v > 0.0).astype(i32), axis=1, keepdims=True)
    k = jnp.minimum(np_i * _NEG_POS_RATIO, P - 1)
    kp = jnp.minimum(k, n_strict)

    def bs_body(_, carry):
        lo, hi = carry
        mid = lo + (hi - lo) // 2
        cnt = jnp.sum((vb > mid).astype(i32), axis=1, keepdims=True)
        go = cnt >= kp
        return jnp.where(go, mid + 1, lo), jnp.where(go, hi, mid)

    lo0 = jnp.zeros((B, 1), i32)
    hi0 = jnp.full((B, 1), 2 ** 31 - 1, i32)
    lo, _ = lax.fori_loop(0, 31, bs_body, (lo0, hi0))
    t = lax.bitcast_convert_type(lo, f32)

    gt_mask = v > t
    c1 = jnp.sum(gt_mask.astype(i32), axis=1, keepdims=True)
    sum_gt = jnp.sum(jnp.where(gt_mask, v, 0.0), axis=1, keepdims=True)
    sum_topk = jnp.where(kp > 0, sum_gt + (kp - c1).astype(f32) * t, 0.0)

    conf_loss = jnp.sum(spce_v + sum_topk)
    num_conf = jnp.sum((np_i + kp).astype(f32))
    loc_loss = jnp.sum(loc_v)
    num_pos4 = jnp.sum(np_v) * 4.0

    o_sub = lax.broadcasted_iota(i32, (8, 128), 0)
    o_lane = lax.broadcasted_iota(i32, (8, 128), 1)
    r0 = o_sub == 0
    out_ref[...] = (
        jnp.where(r0 & (o_lane == 0), loc_loss, 0.0)
        + jnp.where(r0 & (o_lane == 1), num_pos4, 0.0)
        + jnp.where(r0 & (o_lane == 2), conf_loss, 0.0)
        + jnp.where(r0 & (o_lane == 3), num_conf, 0.0)
    )


def kernel(loc_pred, conf_pred, priors, gt_boxes, gt_labels):
    B, P, C = conf_pred.shape
    G = gt_boxes.shape[1]
    pad = _PPAD - P

    loc_t = jnp.pad(
        jnp.transpose(loc_pred, (0, 2, 1)), ((0, 0), (0, 0), (0, pad))
    ).reshape(B, 4, _SUB, _LANE)
    conf_t = jnp.pad(
        jnp.transpose(conf_pred, (0, 2, 1)), ((0, 0), (0, 0), (0, pad))
    ).reshape(B, C, _SUB, _LANE)
    pri_t = jnp.pad(
        priors.T, ((0, 0), (0, pad)), constant_values=1.0
    ).reshape(4, _SUB, _LANE)
    gtl = gt_labels.astype(jnp.int32).reshape(B, 1, G)

    negs, scals = pl.pallas_call(
        functools.partial(_stage1_kernel, P=P, G=G, C=C),
        grid=(B,),
        in_specs=[
            pl.BlockSpec((1, 4, _SUB, _LANE), lambda b: (b, 0, 0, 0)),
            pl.BlockSpec((1, C, _SUB, _LANE), lambda b: (b, 0, 0, 0)),
            pl.BlockSpec((4, _SUB, _LANE), lambda b: (0, 0, 0)),
            pl.BlockSpec((1, G, 4), lambda b: (b, 0, 0)),
            pl.BlockSpec((1, 1, G), lambda b: (b, 0, 0)),
        ],
        out_specs=[
            pl.BlockSpec((1, _SUB, _LANE), lambda b: (b, 0, 0)),
            pl.BlockSpec((1, 1, 128), lambda b: (b, 0, 0)),
        ],
        out_shape=[
            jax.ShapeDtypeStruct((B, _SUB, _LANE), jnp.float32),
            jax.ShapeDtypeStruct((B, 1, 128), jnp.float32),
        ],
    )(loc_t, conf_t, pri_t, gt_boxes, gtl)

    out = pl.pallas_call(
        functools.partial(_stage2_kernel, B=B, P=P),
        grid=(1,),
        in_specs=[
            pl.BlockSpec((B, _PPAD), lambda i: (0, 0)),
            pl.BlockSpec((B, 128), lambda i: (0, 0)),
        ],
        out_specs=pl.BlockSpec((8, 128), lambda i: (0, 0)),
        out_shape=jax.ShapeDtypeStruct((8, 128), jnp.float32),
    )(negs.reshape(B, _PPAD), scals.reshape(B, 128))
    return out[0, 0] / out[0, 1] + out[0, 2] / out[0, 3]


# speedup vs baseline: 35.3184x; 1.6241x over previous
import functools

import jax
import jax.numpy as jnp
from jax import lax
from jax.experimental import pallas as pl

_SUB = 8
_LANE = 1152
_PPAD = _SUB * _LANE

_THRESHOLD = 0.5
_NEG_POS_RATIO = 3
_VAR0 = 0.1
_VAR1 = 0.2


def _stage1_kernel(loc_ref, conf_ref, pri_ref, gtb_ref, gtl_ref,
                   neg_ref, scal_ref, *, P, G, C):
    f32 = jnp.float32
    i32 = jnp.int32

    sub_i = lax.broadcasted_iota(i32, (_SUB, _LANE), 0)
    lane_i = lax.broadcasted_iota(i32, (_SUB, _LANE), 1)
    pid = sub_i * _LANE + lane_i
    valid = pid < P

    pri = pri_ref[...]
    pcx, pcy, pw, ph = pri[0], pri[1], pri[2], pri[3]
    px1 = pcx - pw / 2.0
    py1 = pcy - ph / 2.0
    px2 = pcx + pw / 2.0
    py2 = pcy + ph / 2.0
    area_p = (px2 - px1) * (py2 - py1)

    max_iou = jnp.full((_SUB, _LANE), -1.0, f32)
    bgpp = jnp.zeros((_SUB, _LANE), i32)
    ious = []
    for g in range(G):
        gx1 = gtb_ref[0, g, 0]
        gy1 = gtb_ref[0, g, 1]
        gx2 = gtb_ref[0, g, 2]
        gy2 = gtb_ref[0, g, 3]
        ltx = jnp.maximum(px1, gx1)
        lty = jnp.maximum(py1, gy1)
        rbx = jnp.minimum(px2, gx2)
        rby = jnp.minimum(py2, gy2)
        wx = jnp.clip(rbx - ltx, 0.0, None)
        wy = jnp.clip(rby - lty, 0.0, None)
        inter = wx * wy
        area_g = (gx2 - gx1) * (gy2 - gy1)
        iou_g = inter / (area_p + area_g - inter + 1e-10)
        iou_g = jnp.where(valid, iou_g, -1.0)
        ious.append(iou_g)
        better = iou_g > max_iou
        bgpp = jnp.where(better, g, bgpp)
        max_iou = jnp.where(better, iou_g, max_iou)

    big = jnp.int32(2 ** 30)
    iou3 = jnp.stack(ious)
    mxv = jnp.max(iou3, axis=(1, 2), keepdims=True)
    cand = jnp.where(iou3 == mxv, pid[None], big)
    bp3 = jnp.min(cand, axis=(1, 2), keepdims=True)
    g_io3 = lax.broadcasted_iota(i32, (G, _SUB, _LANE), 0)
    forced = jnp.max(jnp.where(pid[None] == bp3, g_io3, -1), axis=0)

    above = max_iou >= _THRESHOLD
    matched = jnp.where(above, jnp.where(forced >= 0, forced, bgpp), -1)
    pos = matched >= 0
    num_pos = jnp.sum(pos.astype(f32), keepdims=True)[None]
    idx = jnp.maximum(matched, 0)

    mx1 = jnp.zeros((_SUB, _LANE), f32)
    my1 = jnp.zeros((_SUB, _LANE), f32)
    mx2 = jnp.zeros((_SUB, _LANE), f32)
    my2 = jnp.zeros((_SUB, _LANE), f32)
    mlab = jnp.zeros((_SUB, _LANE), f32)
    for g in range(G):
        m = idx == g
        mx1 = jnp.where(m, gtb_ref[0, g, 0], mx1)
        my1 = jnp.where(m, gtb_ref[0, g, 1], my1)
        mx2 = jnp.where(m, gtb_ref[0, g, 2], mx2)
        my2 = jnp.where(m, gtb_ref[0, g, 3], my2)
        mlab = jnp.where(m, gtl_ref[0, 0, g].astype(f32), mlab)

    bcx = (mx1 + mx2) / 2.0
    bcy = (my1 + my2) / 2.0
    bw = mx2 - mx1
    bh = my2 - my1
    g_cx = (bcx - pcx) / (_VAR0 * pw)
    g_cy = (bcy - pcy) / (_VAR0 * ph)
    w_safe = jnp.where(pos, bw, 1.0)
    h_safe = jnp.where(pos, bh, 1.0)
    g_w = jnp.log(jnp.maximum(w_safe / pw, 1e-10)) / _VAR1
    g_h = jnp.log(jnp.maximum(h_safe / ph, 1e-10)) / _VAR1

    loc = loc_ref[0]
    sl1_acc = jnp.zeros((_SUB, _LANE), f32)
    for t, lrow in ((g_cx, loc[0]), (g_cy, loc[1]), (g_w, loc[2]), (g_h, loc[3])):
        d = lrow - jnp.where(pos, t, 0.0)
        ad = jnp.abs(d)
        sl1 = jnp.where(ad < 1.0, 0.5 * d * d, ad - 0.5)
        sl1_acc = sl1_acc + sl1
    loc_loss = jnp.sum(jnp.where(pos, sl1_acc, 0.0), keepdims=True)[None]

    tgt = jnp.where(pos, mlab.astype(i32) - 1, 0)
    conf = conf_ref[0]
    cmax = conf[0]
    for c in range(1, C):
        cmax = jnp.maximum(cmax, conf[c])
    s = jnp.zeros((_SUB, _LANE), f32)
    x_tgt = jnp.zeros((_SUB, _LANE), f32)
    for c in range(C):
        s = s + jnp.exp(conf[c] - cmax)
        x_tgt = jnp.where(tgt == c, conf[c], x_tgt)
    ce = cmax + jnp.log(s) - x_tgt

    sum_pos_ce = jnp.sum(jnp.where(pos, ce, 0.0), keepdims=True)[None]
    neg_loss = jnp.where(pos | (~valid), 0.0, ce)

    neg_ref[...] = neg_loss.reshape(1, _SUB, _LANE)
    o_lane = lax.broadcasted_iota(i32, (1, 1, 128), 2)
    scal_ref[...] = (
        jnp.where(o_lane == 0, loc_loss, 0.0)
        + jnp.where(o_lane == 1, num_pos, 0.0)
        + jnp.where(o_lane == 2, sum_pos_ce, 0.0)
    )


def _stage2_kernel(neg_ref, scal_ref, out_ref, *, B, P):
    f32 = jnp.float32
    i32 = jnp.int32

    v = neg_ref[...]
    vb = lax.bitcast_convert_type(v, i32)
    scal = scal_ref[...]
    loc_v = scal[:, 0:1]
    np_v = scal[:, 1:2]
    spce_v = scal[:, 2:3]

    np_i = np_v.astype(i32)
    n_strict = jnp.sum((v > 0.0).astype(i32), axis=1, keepdims=True)
    k = jnp.minimum(np_i * _NEG_POS_RATIO, P - 1)
    kp = jnp.minimum(k, n_strict)

    def bs_body(_, carry):
        lo, hi = carry
        mid = lo + (hi - lo) // 2
        cnt = jnp.sum((vb > mid).astype(i32), axis=1, keepdims=True)
        go = cnt >= kp
        return jnp.where(go, mid + 1, lo), jnp.where(go, hi, mid)

    lo0 = jnp.zeros((B, 1), i32)
    hi0 = jnp.full((B, 1), 2 ** 31 - 1, i32)
    lo, _ = lax.fori_loop(0, 31, bs_body, (lo0, hi0))
    t = lax.bitcast_convert_type(lo, f32)

    gt_mask = v > t
    c1 = jnp.sum(gt_mask.astype(i32), axis=1, keepdims=True)
    sum_gt = jnp.sum(jnp.where(gt_mask, v, 0.0), axis=1, keepdims=True)
    sum_topk = jnp.where(kp > 0, sum_gt + (kp - c1).astype(f32) * t, 0.0)

    conf_loss = jnp.sum(spce_v + sum_topk)
    num_conf = jnp.sum((np_i + kp).astype(f32))
    loc_loss = jnp.sum(loc_v)
    num_pos4 = jnp.sum(np_v) * 4.0

    o_sub = lax.broadcasted_iota(i32, (8, 128), 0)
    o_lane = lax.broadcasted_iota(i32, (8, 128), 1)
    r0 = o_sub == 0
    out_ref[...] = (
        jnp.where(r0 & (o_lane == 0), loc_loss, 0.0)
        + jnp.where(r0 & (o_lane == 1), num_pos4, 0.0)
        + jnp.where(r0 & (o_lane == 2), conf_loss, 0.0)
        + jnp.where(r0 & (o_lane == 3), num_conf, 0.0)
    )


def kernel(loc_pred, conf_pred, priors, gt_boxes, gt_labels):
    B, P, C = conf_pred.shape
    G = gt_boxes.shape[1]
    pad = _PPAD - P

    loc_t = jnp.pad(
        jnp.transpose(loc_pred, (0, 2, 1)), ((0, 0), (0, 0), (0, pad))
    ).reshape(B, 4, _SUB, _LANE)
    conf_t = jnp.pad(
        jnp.transpose(conf_pred, (0, 2, 1)), ((0, 0), (0, 0), (0, pad))
    ).reshape(B, C, _SUB, _LANE)
    pri_t = jnp.pad(
        priors.T, ((0, 0), (0, pad)), constant_values=1.0
    ).reshape(4, _SUB, _LANE)
    gtl = gt_labels.astype(jnp.int32).reshape(B, 1, G)

    negs, scals = pl.pallas_call(
        functools.partial(_stage1_kernel, P=P, G=G, C=C),
        grid=(B,),
        in_specs=[
            pl.BlockSpec((1, 4, _SUB, _LANE), lambda b: (b, 0, 0, 0)),
            pl.BlockSpec((1, C, _SUB, _LANE), lambda b: (b, 0, 0, 0)),
            pl.BlockSpec((4, _SUB, _LANE), lambda b: (0, 0, 0)),
            pl.BlockSpec((1, G, 4), lambda b: (b, 0, 0)),
            pl.BlockSpec((1, 1, G), lambda b: (b, 0, 0)),
        ],
        out_specs=[
            pl.BlockSpec((1, _SUB, _LANE), lambda b: (b, 0, 0)),
            pl.BlockSpec((1, 1, 128), lambda b: (b, 0, 0)),
        ],
        out_shape=[
            jax.ShapeDtypeStruct((B, _SUB, _LANE), jnp.float32),
            jax.ShapeDtypeStruct((B, 1, 128), jnp.float32),
        ],
    )(loc_t, conf_t, pri_t, gt_boxes, gtl)

    out = pl.pallas_call(
        functools.partial(_stage2_kernel, B=B, P=P),
        grid=(1,),
        in_specs=[
            pl.BlockSpec((B, _PPAD), lambda i: (0, 0)),
            pl.BlockSpec((B, 128), lambda i: (0, 0)),
        ],
        out_specs=pl.BlockSpec((8, 128), lambda i: (0, 0)),
        out_shape=jax.ShapeDtypeStruct((8, 128), jnp.float32),
    )(negs.reshape(B, _PPAD), scals.reshape(B, 128))
    return out[0, 0] / out[0, 1] + out[0, 2] / out[0, 3]


# speedup vs baseline: 35.3363x; 1.0005x over previous
import functools

import jax
import jax.numpy as jnp
from jax import lax
from jax.experimental import pallas as pl

_SUB = 8
_LANE = 1152
_PPAD = _SUB * _LANE

_THRESHOLD = 0.5
_NEG_POS_RATIO = 3
_VAR0 = 0.1
_VAR1 = 0.2


def _stage1_kernel(loc_ref, conf_ref, pri_ref, gtb_ref, gtl_ref,
                   neg_ref, scal_ref, *, P, G, C):
    f32 = jnp.float32
    i32 = jnp.int32

    sub_i = lax.broadcasted_iota(i32, (_SUB, _LANE), 0)
    lane_i = lax.broadcasted_iota(i32, (_SUB, _LANE), 1)
    pid = sub_i * _LANE + lane_i
    valid = pid < P

    pri = pri_ref[...]
    pcx, pcy, pw, ph = pri[0], pri[1], pri[2], pri[3]
    px1 = pcx - pw / 2.0
    py1 = pcy - ph / 2.0
    px2 = pcx + pw / 2.0
    py2 = pcy + ph / 2.0
    area_p = (px2 - px1) * (py2 - py1)

    max_iou = jnp.full((_SUB, _LANE), -1.0, f32)
    bgpp = jnp.zeros((_SUB, _LANE), i32)
    ious = []
    for g in range(G):
        gx1 = gtb_ref[0, g, 0]
        gy1 = gtb_ref[0, g, 1]
        gx2 = gtb_ref[0, g, 2]
        gy2 = gtb_ref[0, g, 3]
        ltx = jnp.maximum(px1, gx1)
        lty = jnp.maximum(py1, gy1)
        rbx = jnp.minimum(px2, gx2)
        rby = jnp.minimum(py2, gy2)
        wx = jnp.clip(rbx - ltx, 0.0, None)
        wy = jnp.clip(rby - lty, 0.0, None)
        inter = wx * wy
        area_g = (gx2 - gx1) * (gy2 - gy1)
        iou_g = inter / (area_p + area_g - inter + 1e-10)
        iou_g = jnp.where(valid, iou_g, -1.0)
        ious.append(iou_g)
        better = iou_g > max_iou
        bgpp = jnp.where(better, g, bgpp)
        max_iou = jnp.where(better, iou_g, max_iou)

    big = jnp.int32(2 ** 30)
    iou3 = jnp.stack(ious)
    mxv = jnp.max(iou3, axis=(1, 2), keepdims=True)
    cand = jnp.where(iou3 == mxv, pid[None], big)
    bp3 = jnp.min(cand, axis=(1, 2), keepdims=True)
    g_io3 = lax.broadcasted_iota(i32, (G, _SUB, _LANE), 0)
    forced = jnp.max(jnp.where(pid[None] == bp3, g_io3, -1), axis=0)

    above = max_iou >= _THRESHOLD
    matched = jnp.where(above, jnp.where(forced >= 0, forced, bgpp), -1)
    pos = matched >= 0
    num_pos = jnp.sum(pos.astype(f32), keepdims=True)[None]
    idx = jnp.maximum(matched, 0)

    mx1 = jnp.zeros((_SUB, _LANE), f32)
    my1 = jnp.zeros((_SUB, _LANE), f32)
    mx2 = jnp.zeros((_SUB, _LANE), f32)
    my2 = jnp.zeros((_SUB, _LANE), f32)
    mlab = jnp.zeros((_SUB, _LANE), f32)
    for g in range(G):
        m = idx == g
        mx1 = jnp.where(m, gtb_ref[0, g, 0], mx1)
        my1 = jnp.where(m, gtb_ref[0, g, 1], my1)
        mx2 = jnp.where(m, gtb_ref[0, g, 2], mx2)
        my2 = jnp.where(m, gtb_ref[0, g, 3], my2)
        mlab = jnp.where(m, gtl_ref[0, 0, g].astype(f32), mlab)

    bcx = (mx1 + mx2) / 2.0
    bcy = (my1 + my2) / 2.0
    bw = mx2 - mx1
    bh = my2 - my1
    g_cx = (bcx - pcx) / (_VAR0 * pw)
    g_cy = (bcy - pcy) / (_VAR0 * ph)
    w_safe = jnp.where(pos, bw, 1.0)
    h_safe = jnp.where(pos, bh, 1.0)
    g_w = jnp.log(jnp.maximum(w_safe / pw, 1e-10)) / _VAR1
    g_h = jnp.log(jnp.maximum(h_safe / ph, 1e-10)) / _VAR1

    loc = loc_ref[0]
    sl1_acc = jnp.zeros((_SUB, _LANE), f32)
    for t, lrow in ((g_cx, loc[0]), (g_cy, loc[1]), (g_w, loc[2]), (g_h, loc[3])):
        d = lrow - jnp.where(pos, t, 0.0)
        ad = jnp.abs(d)
        sl1 = jnp.where(ad < 1.0, 0.5 * d * d, ad - 0.5)
        sl1_acc = sl1_acc + sl1
    loc_loss = jnp.sum(jnp.where(pos, sl1_acc, 0.0), keepdims=True)[None]

    tgt = jnp.where(pos, mlab.astype(i32) - 1, 0)
    conf = conf_ref[0]
    cmax = conf[0]
    for c in range(1, C):
        cmax = jnp.maximum(cmax, conf[c])
    s = jnp.zeros((_SUB, _LANE), f32)
    x_tgt = jnp.zeros((_SUB, _LANE), f32)
    for c in range(C):
        s = s + jnp.exp(conf[c] - cmax)
        x_tgt = jnp.where(tgt == c, conf[c], x_tgt)
    ce = cmax + jnp.log(s) - x_tgt

    sum_pos_ce = jnp.sum(jnp.where(pos, ce, 0.0), keepdims=True)[None]
    neg_loss = jnp.where(pos | (~valid), 0.0, ce)

    neg_ref[...] = neg_loss.reshape(1, _SUB, _LANE)
    o_lane = lax.broadcasted_iota(i32, (1, 1, 128), 2)
    scal_ref[...] = (
        jnp.where(o_lane == 0, loc_loss, 0.0)
        + jnp.where(o_lane == 1, num_pos, 0.0)
        + jnp.where(o_lane == 2, sum_pos_ce, 0.0)
    )


def _stage2_kernel(neg_ref, scal_ref, out_ref, *, B, P):
    f32 = jnp.float32
    i32 = jnp.int32

    v = neg_ref[...]
    vb = lax.bitcast_convert_type(v, i32)
    scal = scal_ref[...]
    loc_v = scal[:, 0:1]
    np_v = scal[:, 1:2]
    spce_v = scal[:, 2:3]

    np_i = np_v.astype(i32)
    n_strict = jnp.sum((v > 0.0).astype(i32), axis=1, keepdims=True)
    k = jnp.minimum(np_i * _NEG_POS_RATIO, P - 1)
    kp = jnp.minimum(k, n_strict)

    def bs_body(_, carry):
        lo, hi = carry
        mid = lo + (hi - lo) // 2
        cnt = jnp.sum((vb > mid).astype(i32), axis=1, keepdims=True)
        go = cnt >= kp
        return jnp.where(go, mid + 1, lo), jnp.where(go, hi, mid)

    lo0 = jnp.zeros((B, 1), i32)
    hi0 = jnp.full((B, 1), 2 ** 31 - 1, i32)
    lo, _ = lax.fori_loop(0, 31, bs_body, (lo0, hi0))
    t = lax.bitcast_convert_type(lo, f32)

    gt_mask = v > t
    c1 = jnp.sum(gt_mask.astype(i32), axis=1, keepdims=True)
    sum_gt = jnp.sum(jnp.where(gt_mask, v, 0.0), axis=1, keepdims=True)
    sum_topk = jnp.where(kp > 0, sum_gt + (kp - c1).astype(f32) * t, 0.0)

    conf_loss = jnp.sum(spce_v + sum_topk)
    num_conf = jnp.sum((np_i + kp).astype(f32))
    loc_loss = jnp.sum(loc_v)
    num_pos4 = jnp.sum(np_v) * 4.0

    o_sub = lax.broadcasted_iota(i32, (8, 128), 0)
    o_lane = lax.broadcasted_iota(i32, (8, 128), 1)
    r0 = o_sub == 0
    out_ref[...] = (
        jnp.where(r0 & (o_lane == 0), loc_loss, 0.0)
        + jnp.where(r0 & (o_lane == 1), num_pos4, 0.0)
        + jnp.where(r0 & (o_lane == 2), conf_loss, 0.0)
        + jnp.where(r0 & (o_lane == 3), num_conf, 0.0)
    )


def kernel(loc_pred, conf_pred, priors, gt_boxes, gt_labels):
    B, P, C = conf_pred.shape
    G = gt_boxes.shape[1]
    pad = _PPAD - P

    loc_t = (
        jnp.zeros((B, 4, _PPAD), jnp.float32)
        .at[:, :, :P].set(jnp.transpose(loc_pred, (0, 2, 1)))
        .reshape(B, 4, _SUB, _LANE)
    )
    conf_t = (
        jnp.zeros((B, C, _PPAD), jnp.float32)
        .at[:, :, :P].set(jnp.transpose(conf_pred, (0, 2, 1)))
        .reshape(B, C, _SUB, _LANE)
    )
    pri_t = jnp.pad(
        priors.T, ((0, 0), (0, pad)), constant_values=1.0
    ).reshape(4, _SUB, _LANE)
    gtl = gt_labels.astype(jnp.int32).reshape(B, 1, G)

    negs, scals = pl.pallas_call(
        functools.partial(_stage1_kernel, P=P, G=G, C=C),
        grid=(B,),
        in_specs=[
            pl.BlockSpec((1, 4, _SUB, _LANE), lambda b: (b, 0, 0, 0)),
            pl.BlockSpec((1, C, _SUB, _LANE), lambda b: (b, 0, 0, 0)),
            pl.BlockSpec((4, _SUB, _LANE), lambda b: (0, 0, 0)),
            pl.BlockSpec((1, G, 4), lambda b: (b, 0, 0)),
            pl.BlockSpec((1, 1, G), lambda b: (b, 0, 0)),
        ],
        out_specs=[
            pl.BlockSpec((1, _SUB, _LANE), lambda b: (b, 0, 0)),
            pl.BlockSpec((1, 1, 128), lambda b: (b, 0, 0)),
        ],
        out_shape=[
            jax.ShapeDtypeStruct((B, _SUB, _LANE), jnp.float32),
            jax.ShapeDtypeStruct((B, 1, 128), jnp.float32),
        ],
    )(loc_t, conf_t, pri_t, gt_boxes, gtl)

    out = pl.pallas_call(
        functools.partial(_stage2_kernel, B=B, P=P),
        grid=(1,),
        in_specs=[
            pl.BlockSpec((B, _PPAD), lambda i: (0, 0)),
            pl.BlockSpec((B, 128), lambda i: (0, 0)),
        ],
        out_specs=pl.BlockSpec((8, 128), lambda i: (0, 0)),
        out_shape=jax.ShapeDtypeStruct((8, 128), jnp.float32),
    )(negs.reshape(B, _PPAD), scals.reshape(B, 128))
    return out[0, 0] / out[0, 1] + out[0, 2] / out[0, 3]


# speedup vs baseline: 35.9185x; 1.0165x over previous
import functools

import jax
import jax.numpy as jnp
from jax import lax
from jax.experimental import pallas as pl
from jax.experimental.pallas import tpu as pltpu

_SUB = 8
_LANE = 1152
_PPAD = _SUB * _LANE

_THRESHOLD = 0.5
_NEG_POS_RATIO = 3
_VAR0 = 0.1
_VAR1 = 0.2


def _fused_kernel(loc_ref, conf_ref, pri_ref, gtb_ref, gtl_ref,
                  out_ref, negs_s, scal_s, *, B, P, G, C):
    f32 = jnp.float32
    i32 = jnp.int32
    b = pl.program_id(0)

    sub_i = lax.broadcasted_iota(i32, (_SUB, _LANE), 0)
    lane_i = lax.broadcasted_iota(i32, (_SUB, _LANE), 1)
    pid = sub_i * _LANE + lane_i
    valid = pid < P

    pri = pri_ref[...]
    pcx, pcy, pw, ph = pri[0], pri[1], pri[2], pri[3]
    px1 = pcx - pw / 2.0
    py1 = pcy - ph / 2.0
    px2 = pcx + pw / 2.0
    py2 = pcy + ph / 2.0
    area_p = (px2 - px1) * (py2 - py1)

    max_iou = jnp.full((_SUB, _LANE), -1.0, f32)
    bgpp = jnp.zeros((_SUB, _LANE), i32)
    ious = []
    for g in range(G):
        gx1 = gtb_ref[0, g, 0]
        gy1 = gtb_ref[0, g, 1]
        gx2 = gtb_ref[0, g, 2]
        gy2 = gtb_ref[0, g, 3]
        ltx = jnp.maximum(px1, gx1)
        lty = jnp.maximum(py1, gy1)
        rbx = jnp.minimum(px2, gx2)
        rby = jnp.minimum(py2, gy2)
        wx = jnp.clip(rbx - ltx, 0.0, None)
        wy = jnp.clip(rby - lty, 0.0, None)
        inter = wx * wy
        area_g = (gx2 - gx1) * (gy2 - gy1)
        iou_g = inter / (area_p + area_g - inter + 1e-10)
        iou_g = jnp.where(valid, iou_g, -1.0)
        ious.append(iou_g)
        better = iou_g > max_iou
        bgpp = jnp.where(better, g, bgpp)
        max_iou = jnp.where(better, iou_g, max_iou)

    big = jnp.int32(2 ** 30)
    iou3 = jnp.stack(ious)
    mxv = jnp.max(iou3, axis=(1, 2), keepdims=True)
    cand = jnp.where(iou3 == mxv, pid[None], big)
    bp3 = jnp.min(cand, axis=(1, 2), keepdims=True)
    g_io3 = lax.broadcasted_iota(i32, (G, _SUB, _LANE), 0)
    forced = jnp.max(jnp.where(pid[None] == bp3, g_io3, -1), axis=0)

    above = max_iou >= _THRESHOLD
    matched = jnp.where(above, jnp.where(forced >= 0, forced, bgpp), -1)
    pos = matched >= 0
    num_pos = jnp.sum(pos.astype(f32), keepdims=True)[None]
    idx = jnp.maximum(matched, 0)

    mx1 = jnp.zeros((_SUB, _LANE), f32)
    my1 = jnp.zeros((_SUB, _LANE), f32)
    mx2 = jnp.zeros((_SUB, _LANE), f32)
    my2 = jnp.zeros((_SUB, _LANE), f32)
    mlab = jnp.zeros((_SUB, _LANE), f32)
    for g in range(G):
        m = idx == g
        mx1 = jnp.where(m, gtb_ref[0, g, 0], mx1)
        my1 = jnp.where(m, gtb_ref[0, g, 1], my1)
        mx2 = jnp.where(m, gtb_ref[0, g, 2], mx2)
        my2 = jnp.where(m, gtb_ref[0, g, 3], my2)
        mlab = jnp.where(m, gtl_ref[0, 0, g].astype(f32), mlab)

    bcx = (mx1 + mx2) / 2.0
    bcy = (my1 + my2) / 2.0
    bw = mx2 - mx1
    bh = my2 - my1
    g_cx = (bcx - pcx) / (_VAR0 * pw)
    g_cy = (bcy - pcy) / (_VAR0 * ph)
    w_safe = jnp.where(pos, bw, 1.0)
    h_safe = jnp.where(pos, bh, 1.0)
    g_w = jnp.log(jnp.maximum(w_safe / pw, 1e-10)) / _VAR1
    g_h = jnp.log(jnp.maximum(h_safe / ph, 1e-10)) / _VAR1

    loc = loc_ref[0]
    sl1_acc = jnp.zeros((_SUB, _LANE), f32)
    for t, lrow in ((g_cx, loc[0]), (g_cy, loc[1]), (g_w, loc[2]), (g_h, loc[3])):
        d = lrow - jnp.where(pos, t, 0.0)
        ad = jnp.abs(d)
        sl1 = jnp.where(ad < 1.0, 0.5 * d * d, ad - 0.5)
        sl1_acc = sl1_acc + sl1
    loc_loss = jnp.sum(jnp.where(pos, sl1_acc, 0.0), keepdims=True)[None]

    tgt = jnp.where(pos, mlab.astype(i32) - 1, 0)
    conf = conf_ref[0]
    cmax = conf[0]
    for c in range(1, C):
        cmax = jnp.maximum(cmax, conf[c])
    s = jnp.zeros((_SUB, _LANE), f32)
    x_tgt = jnp.zeros((_SUB, _LANE), f32)
    for c in range(C):
        s = s + jnp.exp(conf[c] - cmax)
        x_tgt = jnp.where(tgt == c, conf[c], x_tgt)
    ce = cmax + jnp.log(s) - x_tgt

    sum_pos_ce = jnp.sum(jnp.where(pos, ce, 0.0), keepdims=True)[None]
    neg_loss = jnp.where(pos | (~valid), 0.0, ce)

    negs_s[pl.ds(b, 1)] = neg_loss.reshape(1, _SUB, _LANE)
    o_lane = lax.broadcasted_iota(i32, (1, 1, 128), 2)
    scal_s[pl.ds(b, 1)] = (
        jnp.where(o_lane == 0, loc_loss, 0.0)
        + jnp.where(o_lane == 1, num_pos, 0.0)
        + jnp.where(o_lane == 2, sum_pos_ce, 0.0)
    )

    @pl.when(b == B - 1)
    def _():
        _mine_and_finish(negs_s, scal_s, out_ref, B, P)


def _mine_and_finish(neg_ref, scal_ref, out_ref, B, P):
    f32 = jnp.float32
    i32 = jnp.int32

    v = neg_ref[...]
    vb = lax.bitcast_convert_type(v, i32)
    scal = scal_ref[...]
    loc_v = scal[:, :, 0:1]
    np_v = scal[:, :, 1:2]
    spce_v = scal[:, :, 2:3]

    np_i = np_v.astype(i32)
    n_strict = jnp.sum((v > 0.0).astype(i32), axis=(1, 2), keepdims=True)
    k = jnp.minimum(np_i * _NEG_POS_RATIO, P - 1)
    kp = jnp.minimum(k, n_strict)

    def bs_body(_, carry):
        lo, hi = carry
        mid = lo + (hi - lo) // 2
        cnt = jnp.sum((vb > mid).astype(i32), axis=(1, 2), keepdims=True)
        go = cnt >= kp
        return jnp.where(go, mid + 1, lo), jnp.where(go, hi, mid)

    lo0 = jnp.zeros((B, 1, 1), i32)
    hi0 = jnp.full((B, 1, 1), 2 ** 31 - 1, i32)
    lo, _ = lax.fori_loop(0, 31, bs_body, (lo0, hi0))
    t = lax.bitcast_convert_type(lo, f32)

    gt_mask = v > t
    c1 = jnp.sum(gt_mask.astype(i32), axis=(1, 2), keepdims=True)
    sum_gt = jnp.sum(jnp.where(gt_mask, v, 0.0), axis=(1, 2), keepdims=True)
    sum_topk = jnp.where(kp > 0, sum_gt + (kp - c1).astype(f32) * t, 0.0)

    conf_loss = jnp.sum(spce_v + sum_topk)
    num_conf = jnp.sum((np_i + kp).astype(f32))
    loc_loss = jnp.sum(loc_v)
    num_pos4 = jnp.sum(np_v) * 4.0

    o_sub = lax.broadcasted_iota(i32, (8, 128), 0)
    o_lane = lax.broadcasted_iota(i32, (8, 128), 1)
    r0 = o_sub == 0
    out_ref[...] = (
        jnp.where(r0 & (o_lane == 0), loc_loss, 0.0)
        + jnp.where(r0 & (o_lane == 1), num_pos4, 0.0)
        + jnp.where(r0 & (o_lane == 2), conf_loss, 0.0)
        + jnp.where(r0 & (o_lane == 3), num_conf, 0.0)
    )


def kernel(loc_pred, conf_pred, priors, gt_boxes, gt_labels):
    B, P, C = conf_pred.shape
    G = gt_boxes.shape[1]
    pad = _PPAD - P

    loc_t = (
        jnp.zeros((B, 4, _PPAD), jnp.float32)
        .at[:, :, :P].set(jnp.transpose(loc_pred, (0, 2, 1)))
        .reshape(B, 4, _SUB, _LANE)
    )
    conf_t = (
        jnp.zeros((B, C, _PPAD), jnp.float32)
        .at[:, :, :P].set(jnp.transpose(conf_pred, (0, 2, 1)))
        .reshape(B, C, _SUB, _LANE)
    )
    pri_t = jnp.pad(
        priors.T, ((0, 0), (0, pad)), constant_values=1.0
    ).reshape(4, _SUB, _LANE)
    gtl = gt_labels.astype(jnp.int32).reshape(B, 1, G)

    out = pl.pallas_call(
        functools.partial(_fused_kernel, B=B, P=P, G=G, C=C),
        grid=(B,),
        in_specs=[
            pl.BlockSpec((1, 4, _SUB, _LANE), lambda b: (b, 0, 0, 0)),
            pl.BlockSpec((1, C, _SUB, _LANE), lambda b: (b, 0, 0, 0)),
            pl.BlockSpec((4, _SUB, _LANE), lambda b: (0, 0, 0)),
            pl.BlockSpec((1, G, 4), lambda b: (b, 0, 0)),
            pl.BlockSpec((1, 1, G), lambda b: (b, 0, 0)),
        ],
        out_specs=pl.BlockSpec((8, 128), lambda b: (0, 0)),
        out_shape=jax.ShapeDtypeStruct((8, 128), jnp.float32),
        scratch_shapes=[
            pltpu.VMEM((B, _SUB, _LANE), jnp.float32),
            pltpu.VMEM((B, 1, 128), jnp.float32),
        ],
    )(loc_t, conf_t, pri_t, gt_boxes, gtl)
    return out[0, 0] / out[0, 1] + out[0, 2] / out[0, 3]


# speedup vs baseline: 36.3482x; 1.0120x over previous
import functools

import jax
import jax.numpy as jnp
from jax import lax
from jax.experimental import pallas as pl
from jax.experimental.pallas import tpu as pltpu

_SUB = 8
_LANE = 1152
_PPAD = _SUB * _LANE

_THRESHOLD = 0.5
_NEG_POS_RATIO = 3
_VAR0 = 0.1
_VAR1 = 0.2


_NIMG = 2


def _fused_kernel(loc_ref, conf_ref, pri_ref, gtb_ref, gtl_ref,
                  out_ref, negs_s, scal_s, *, B, P, G, C):
    b = pl.program_id(0)
    for j in range(_NIMG):
        _process_image(j, b, loc_ref, conf_ref, pri_ref, gtb_ref, gtl_ref,
                       negs_s, scal_s, P, G, C)

    @pl.when(b == (B // _NIMG) - 1)
    def _():
        _mine_and_finish(negs_s, scal_s, out_ref, B, P)


def _process_image(j, b, loc_ref, conf_ref, pri_ref, gtb_ref, gtl_ref,
                   negs_s, scal_s, P, G, C):
    f32 = jnp.float32
    i32 = jnp.int32

    sub_i = lax.broadcasted_iota(i32, (_SUB, _LANE), 0)
    lane_i = lax.broadcasted_iota(i32, (_SUB, _LANE), 1)
    pid = sub_i * _LANE + lane_i
    valid = pid < P

    pri = pri_ref[...]
    pcx, pcy, pw, ph = pri[0], pri[1], pri[2], pri[3]
    px1 = pcx - pw / 2.0
    py1 = pcy - ph / 2.0
    px2 = pcx + pw / 2.0
    py2 = pcy + ph / 2.0
    area_p = (px2 - px1) * (py2 - py1)

    max_iou = jnp.full((_SUB, _LANE), -1.0, f32)
    bgpp = jnp.zeros((_SUB, _LANE), i32)
    ious = []
    for g in range(G):
        gx1 = gtb_ref[j, g, 0]
        gy1 = gtb_ref[j, g, 1]
        gx2 = gtb_ref[j, g, 2]
        gy2 = gtb_ref[j, g, 3]
        ltx = jnp.maximum(px1, gx1)
        lty = jnp.maximum(py1, gy1)
        rbx = jnp.minimum(px2, gx2)
        rby = jnp.minimum(py2, gy2)
        wx = jnp.clip(rbx - ltx, 0.0, None)
        wy = jnp.clip(rby - lty, 0.0, None)
        inter = wx * wy
        area_g = (gx2 - gx1) * (gy2 - gy1)
        iou_g = inter / (area_p + area_g - inter + 1e-10)
        iou_g = jnp.where(valid, iou_g, -1.0)
        ious.append(iou_g)
        better = iou_g > max_iou
        bgpp = jnp.where(better, g, bgpp)
        max_iou = jnp.where(better, iou_g, max_iou)

    big = jnp.int32(2 ** 30)
    iou3 = jnp.stack(ious)
    mxv = jnp.max(iou3, axis=(1, 2), keepdims=True)
    cand = jnp.where(iou3 == mxv, pid[None], big)
    bp3 = jnp.min(cand, axis=(1, 2), keepdims=True)
    g_io3 = lax.broadcasted_iota(i32, (G, _SUB, _LANE), 0)
    forced = jnp.max(jnp.where(pid[None] == bp3, g_io3, -1), axis=0)

    above = max_iou >= _THRESHOLD
    matched = jnp.where(above, jnp.where(forced >= 0, forced, bgpp), -1)
    pos = matched >= 0
    num_pos = jnp.sum(pos.astype(f32), keepdims=True)[None]
    idx = jnp.maximum(matched, 0)

    mx1 = jnp.zeros((_SUB, _LANE), f32)
    my1 = jnp.zeros((_SUB, _LANE), f32)
    mx2 = jnp.zeros((_SUB, _LANE), f32)
    my2 = jnp.zeros((_SUB, _LANE), f32)
    mlab = jnp.zeros((_SUB, _LANE), f32)
    for g in range(G):
        m = idx == g
        mx1 = jnp.where(m, gtb_ref[j, g, 0], mx1)
        my1 = jnp.where(m, gtb_ref[j, g, 1], my1)
        mx2 = jnp.where(m, gtb_ref[j, g, 2], mx2)
        my2 = jnp.where(m, gtb_ref[j, g, 3], my2)
        mlab = jnp.where(m, gtl_ref[j, 0, g].astype(f32), mlab)

    bcx = (mx1 + mx2) / 2.0
    bcy = (my1 + my2) / 2.0
    bw = mx2 - mx1
    bh = my2 - my1
    g_cx = (bcx - pcx) / (_VAR0 * pw)
    g_cy = (bcy - pcy) / (_VAR0 * ph)
    w_safe = jnp.where(pos, bw, 1.0)
    h_safe = jnp.where(pos, bh, 1.0)
    g_w = jnp.log(jnp.maximum(w_safe / pw, 1e-10)) / _VAR1
    g_h = jnp.log(jnp.maximum(h_safe / ph, 1e-10)) / _VAR1

    loc = loc_ref[j]
    sl1_acc = jnp.zeros((_SUB, _LANE), f32)
    for t, lrow in ((g_cx, loc[0]), (g_cy, loc[1]), (g_w, loc[2]), (g_h, loc[3])):
        d = lrow - jnp.where(pos, t, 0.0)
        ad = jnp.abs(d)
        sl1 = jnp.where(ad < 1.0, 0.5 * d * d, ad - 0.5)
        sl1_acc = sl1_acc + sl1
    loc_loss = jnp.sum(jnp.where(pos, sl1_acc, 0.0), keepdims=True)[None]

    tgt = jnp.where(pos, mlab.astype(i32) - 1, 0)
    conf = conf_ref[j]
    cmax = conf[0]
    for c in range(1, C):
        cmax = jnp.maximum(cmax, conf[c])
    s = jnp.zeros((_SUB, _LANE), f32)
    x_tgt = jnp.zeros((_SUB, _LANE), f32)
    for c in range(C):
        s = s + jnp.exp(conf[c] - cmax)
        x_tgt = jnp.where(tgt == c, conf[c], x_tgt)
    ce = cmax + jnp.log(s) - x_tgt

    sum_pos_ce = jnp.sum(jnp.where(pos, ce, 0.0), keepdims=True)[None]
    neg_loss = jnp.where(pos | (~valid), 0.0, ce)

    bi = b * _NIMG + j
    negs_s[pl.ds(bi, 1)] = neg_loss.reshape(1, _SUB, _LANE)
    o_lane = lax.broadcasted_iota(i32, (1, 1, 128), 2)
    scal_s[pl.ds(bi, 1)] = (
        jnp.where(o_lane == 0, loc_loss, 0.0)
        + jnp.where(o_lane == 1, num_pos, 0.0)
        + jnp.where(o_lane == 2, sum_pos_ce, 0.0)
    )


def _mine_and_finish(neg_ref, scal_ref, out_ref, B, P):
    f32 = jnp.float32
    i32 = jnp.int32

    v = neg_ref[...]
    vb = lax.bitcast_convert_type(v, i32)
    scal = scal_ref[...]
    loc_v = scal[:, :, 0:1]
    np_v = scal[:, :, 1:2]
    spce_v = scal[:, :, 2:3]

    np_i = np_v.astype(i32)
    n_strict = jnp.sum((v > 0.0).astype(i32), axis=(1, 2), keepdims=True)
    k = jnp.minimum(np_i * _NEG_POS_RATIO, P - 1)
    kp = jnp.minimum(k, n_strict)

    def bs_body(_, carry):
        lo, hi = carry
        mid = lo + (hi - lo) // 2
        cnt = jnp.sum((vb > mid).astype(i32), axis=(1, 2), keepdims=True)
        go = cnt >= kp
        return jnp.where(go, mid + 1, lo), jnp.where(go, hi, mid)

    lo0 = jnp.zeros((B, 1, 1), i32)
    hi0 = jnp.full((B, 1, 1), 2 ** 31 - 1, i32)
    lo, _ = lax.fori_loop(0, 31, bs_body, (lo0, hi0))
    t = lax.bitcast_convert_type(lo, f32)

    gt_mask = v > t
    c1 = jnp.sum(gt_mask.astype(i32), axis=(1, 2), keepdims=True)
    sum_gt = jnp.sum(jnp.where(gt_mask, v, 0.0), axis=(1, 2), keepdims=True)
    sum_topk = jnp.where(kp > 0, sum_gt + (kp - c1).astype(f32) * t, 0.0)

    conf_loss = jnp.sum(spce_v + sum_topk)
    num_conf = jnp.sum((np_i + kp).astype(f32))
    loc_loss = jnp.sum(loc_v)
    num_pos4 = jnp.sum(np_v) * 4.0

    o_sub = lax.broadcasted_iota(i32, (8, 128), 0)
    o_lane = lax.broadcasted_iota(i32, (8, 128), 1)
    r0 = o_sub == 0
    out_ref[...] = (
        jnp.where(r0 & (o_lane == 0), loc_loss, 0.0)
        + jnp.where(r0 & (o_lane == 1), num_pos4, 0.0)
        + jnp.where(r0 & (o_lane == 2), conf_loss, 0.0)
        + jnp.where(r0 & (o_lane == 3), num_conf, 0.0)
    )


def kernel(loc_pred, conf_pred, priors, gt_boxes, gt_labels):
    B, P, C = conf_pred.shape
    G = gt_boxes.shape[1]
    pad = _PPAD - P

    loc_t = (
        jnp.zeros((B, 4, _PPAD), jnp.float32)
        .at[:, :, :P].set(jnp.transpose(loc_pred, (0, 2, 1)))
        .reshape(B, 4, _SUB, _LANE)
    )
    conf_t = (
        jnp.zeros((B, C, _PPAD), jnp.float32)
        .at[:, :, :P].set(jnp.transpose(conf_pred, (0, 2, 1)))
        .reshape(B, C, _SUB, _LANE)
    )
    pri_t = jnp.pad(
        priors.T, ((0, 0), (0, pad)), constant_values=1.0
    ).reshape(4, _SUB, _LANE)
    gtl = gt_labels.astype(jnp.int32).reshape(B, 1, G)

    out = pl.pallas_call(
        functools.partial(_fused_kernel, B=B, P=P, G=G, C=C),
        grid=(B // _NIMG,),
        in_specs=[
            pl.BlockSpec((_NIMG, 4, _SUB, _LANE), lambda b: (b, 0, 0, 0)),
            pl.BlockSpec((_NIMG, C, _SUB, _LANE), lambda b: (b, 0, 0, 0)),
            pl.BlockSpec((4, _SUB, _LANE), lambda b: (0, 0, 0)),
            pl.BlockSpec((_NIMG, G, 4), lambda b: (b, 0, 0)),
            pl.BlockSpec((_NIMG, 1, G), lambda b: (b, 0, 0)),
        ],
        out_specs=pl.BlockSpec((8, 128), lambda b: (0, 0)),
        out_shape=jax.ShapeDtypeStruct((8, 128), jnp.float32),
        scratch_shapes=[
            pltpu.VMEM((B, _SUB, _LANE), jnp.float32),
            pltpu.VMEM((B, 1, 128), jnp.float32),
        ],
    )(loc_t, conf_t, pri_t, gt_boxes, gtl)
    return out[0, 0] / out[0, 1] + out[0, 2] / out[0, 3]
